# baseline (speedup 1.0000x reference)
import functools

import jax
import jax.numpy as jnp
from jax import lax
from jax.experimental import pallas as pl
from jax.experimental.pallas import tpu as pltpu
from jax.experimental.pallas import tpu_sc as plsc

B = 2048
D = 256
R = 256
NB = B // R
GAMMA = 0.8
EPS = 1e-14
RHO = 8.0
TAU_INIT = 0.01
BETA_U = 0.5
GRAD_CLIP = 5.0
ETA_INIT = 1e-05


def _side_body(feat_ref, otherT_ref, orows_ref, tau_ref, s_ref, b_ref, e0_ref,
               loss_ref, twsum_ref, twmax_ref, twmin_ref, tausum_ref):
    i = pl.program_id(0)
    feat = feat_ref[...]
    otherT = otherT_ref[...]
    S = jnp.dot(feat, otherT, preferred_element_type=jnp.float32)
    d = jnp.sum(feat * orows_ref[...], axis=1)
    tau = tau_ref[0, 0, :]
    s_old = s_ref[0, 0, :]
    b_old = b_ref[0, 0, :]
    rtau = 1.0 / tau
    m = jnp.max(S, axis=1)
    b_new = jnp.maximum(b_old, (m - d) * rtau)
    c2 = d * rtau + b_new
    E = jnp.exp(S * rtau[:, None] - c2[:, None])
    gwd = jnp.sum(E, axis=1)
    sES = jnp.sum(E * S, axis=1)
    g = gwd - jnp.exp(-b_new)
    P1 = sES - d * gwd
    ema = (1.0 - GAMMA) * s_old * jnp.exp(b_old - b_new) + GAMMA * g
    e0 = e0_ref[0, 0]
    sI = e0 * g + (1.0 - e0) * ema
    sIc = jnp.maximum(sI, EPS)
    rs = 1.0 / sIc
    loss_rows = P1 * rs
    wid_rows = loss_rows * rtau
    tw = jnp.log(sIc / (B - 1)) + b_new + RHO - wid_rows
    tw = jnp.clip(tw, -GRAD_CLIP, GRAD_CLIP)

    blk_loss = jnp.sum(loss_rows)
    blk_twsum = jnp.sum(tw)
    blk_twmax = jnp.max(tw)
    blk_twmin = jnp.min(tw)
    blk_tau = jnp.sum(tau)

    @pl.when(i == 0)
    def _init():
        loss_ref[0, 0] = blk_loss
        twsum_ref[0, 0] = blk_twsum
        twmax_ref[0, 0] = blk_twmax
        twmin_ref[0, 0] = blk_twmin
        tausum_ref[0, 0] = blk_tau

    @pl.when(i != 0)
    def _acc():
        loss_ref[0, 0] += blk_loss
        twsum_ref[0, 0] += blk_twsum
        twmax_ref[0, 0] = jnp.maximum(twmax_ref[0, 0], blk_twmax)
        twmin_ref[0, 0] = jnp.minimum(twmin_ref[0, 0], blk_twmin)
        tausum_ref[0, 0] += blk_tau


_scal = jax.ShapeDtypeStruct((1, 1), jnp.float32)

try:
    _SC_INFO = plsc.get_sparse_core_info()
    _NC, _NS = _SC_INFO.num_cores, _SC_INFO.num_subcores
except ValueError:
    _NC, _NS = 2, 16
_NW = _NC * _NS
_BPW = B // _NW

_vecf = jax.ShapeDtypeStruct((B,), jnp.float32)


@functools.partial(
    pl.kernel,
    mesh=plsc.VectorSubcoreMesh(core_axis_name="c", subcore_axis_name="s",
                                num_cores=_NC, num_subcores=_NS),
    out_type=[_vecf] * 6,
    scratch_types=[
        pltpu.VMEM((_BPW,), jnp.int32),
        pltpu.VMEM((_BPW,), jnp.int32),
        pltpu.VMEM((_BPW,), jnp.float32),
        pltpu.SemaphoreType.DMA,
    ],
)
def _gather6(img_ids, txt_ids, tau_i_t, s_i_t, b_i_t, tau_t_t, s_t_t, b_t_t,
             o_tau_i, o_s_i, o_b_i, o_tau_t, o_s_t, o_b_t,
             idx_i, idx_t, buf, sem):
    wid = lax.axis_index("s") * _NC + lax.axis_index("c")
    base = wid * _BPW
    pltpu.sync_copy(img_ids.at[pl.ds(base, _BPW)], idx_i)
    pltpu.sync_copy(txt_ids.at[pl.ds(base, _BPW)], idx_t)
    for table, idx, out in ((tau_i_t, idx_i, o_tau_i), (s_i_t, idx_i, o_s_i),
                            (b_i_t, idx_i, o_b_i), (tau_t_t, idx_t, o_tau_t),
                            (s_t_t, idx_t, o_s_t), (b_t_t, idx_t, o_b_t)):
        pltpu.async_copy(table.at[idx], buf, sem).wait()
        pltpu.sync_copy(buf, out.at[pl.ds(base, _BPW)])


def _side(feat, otherT, other, tau_g, s_g, b_g, e0, interpret=False):
    tau3 = tau_g.reshape(NB, 1, R)
    s3 = s_g.reshape(NB, 1, R)
    b3 = b_g.reshape(NB, 1, R)
    smem = pltpu.MemorySpace.SMEM
    out = pl.pallas_call(
        _side_body,
        grid=(NB,),
        in_specs=[
            pl.BlockSpec((R, D), lambda i: (i, 0)),
            pl.BlockSpec((D, B), lambda i: (0, 0)),
            pl.BlockSpec((R, D), lambda i: (i, 0)),
            pl.BlockSpec((1, 1, R), lambda i: (i, 0, 0)),
            pl.BlockSpec((1, 1, R), lambda i: (i, 0, 0)),
            pl.BlockSpec((1, 1, R), lambda i: (i, 0, 0)),
            pl.BlockSpec(memory_space=smem),
        ],
        out_specs=[pl.BlockSpec((1, 1), lambda i: (0, 0), memory_space=smem)] * 5,
        out_shape=[_scal] * 5,
        interpret=interpret,
    )(feat, otherT, other, tau3, s3, b3, e0)
    return out


def kernel(image_features, text_features, image_ids, text_ids, epoch, max_epoch,
           s_I, s_T, b_I, b_T, u_I, u_T, tau_I, tau_T, mask_neg):
    tau_i, s_i, b_i, tau_t, s_t, b_t = _gather6(
        image_ids, text_ids, tau_I, s_I, b_I, tau_T, s_T, b_T)

    e0 = (jnp.asarray(epoch) == 0).astype(jnp.float32).reshape(1, 1)
    txtT = text_features.T
    imgT = image_features.T

    li, twi_s, twi_mx, twi_mn, tau_si = _side(image_features, txtT, text_features,
                                              tau_i, s_i, b_i, e0)
    lt, twt_s, _twt_mx, _twt_mn, tau_st = _side(text_features, imgT, image_features,
                                                tau_t, s_t, b_t, e0)

    invB = jnp.float32(1.0 / B)
    total_loss = (li[0, 0] + lt[0, 0]) * invB
    return (total_loss,
            tau_si[0, 0] * invB,
            tau_st[0, 0] * invB,
            jnp.float32(ETA_INIT),
            twi_s[0, 0] * invB,
            twt_s[0, 0] * invB,
            twi_mx[0, 0],
            twi_mn[0, 0])


# speedup vs baseline: 1.1006x; 1.1006x over previous
import functools

import jax
import jax.numpy as jnp
from jax import lax
from jax.experimental import pallas as pl
from jax.experimental.pallas import tpu as pltpu
from jax.experimental.pallas import tpu_sc as plsc

B = 2048
D = 256
R = 256
NB = B // R
GAMMA = 0.8
EPS = 1e-14
RHO = 8.0
GRAD_CLIP = 5.0
ETA_INIT = 1e-05

try:
    _SC_INFO = plsc.get_sparse_core_info()
    _NC, _NS = _SC_INFO.num_cores, _SC_INFO.num_subcores
except ValueError:
    _NC, _NS = 2, 16
_NW = _NC * _NS
_BPW = B // _NW

_vec2 = jax.ShapeDtypeStruct((2, B), jnp.float32)


@functools.partial(
    pl.kernel,
    mesh=plsc.VectorSubcoreMesh(core_axis_name="c", subcore_axis_name="s",
                                num_cores=_NC, num_subcores=_NS),
    out_type=[_vec2] * 3,
    scratch_types=[
        pltpu.VMEM((_BPW,), jnp.int32),
        pltpu.VMEM((_BPW,), jnp.int32),
        pltpu.VMEM((_BPW,), jnp.float32),
        pltpu.SemaphoreType.DMA,
    ],
)
def _gather6(img_ids, txt_ids, tau_i_t, s_i_t, b_i_t, tau_t_t, s_t_t, b_t_t,
             o_tau, o_s, o_b, idx_i, idx_t, buf, sem):
    wid = lax.axis_index("s") * _NC + lax.axis_index("c")
    base = wid * _BPW
    pltpu.sync_copy(img_ids.at[pl.ds(base, _BPW)], idx_i)
    pltpu.sync_copy(txt_ids.at[pl.ds(base, _BPW)], idx_t)
    for row, idx, table, out in ((0, idx_i, tau_i_t, o_tau), (0, idx_i, s_i_t, o_s),
                                 (0, idx_i, b_i_t, o_b), (1, idx_t, tau_t_t, o_tau),
                                 (1, idx_t, s_t_t, o_s), (1, idx_t, b_t_t, o_b)):
        pltpu.async_copy(table.at[idx], buf, sem).wait()
        pltpu.sync_copy(buf, out.at[row, pl.ds(base, _BPW)])


def _tc_body(img_ref, txt_ref, OT_ref, tau_ref, s_ref, b_ref, e0_ref,
             loss_ref, taui_ref, taut_ref, twim_ref, twtm_ref,
             twmax_ref, twmin_ref, acc):
    s = pl.program_id(0)
    i = pl.program_id(1)
    is_img = s == 0
    img_blk = img_ref[...]
    txt_blk = txt_ref[...]
    feat = jnp.where(is_img, img_blk, txt_blk)
    orows = jnp.where(is_img, txt_blk, img_blk)
    OT = OT_ref[...]
    S = jnp.dot(feat, OT, preferred_element_type=jnp.float32)
    d = jnp.sum(feat * orows, axis=1)
    tau = tau_ref[0, 0, :]
    s_old = s_ref[0, 0, :]
    b_old = b_ref[0, 0, :]
    rtau = 1.0 / tau
    m = jnp.max(S, axis=1)
    b_new = jnp.maximum(b_old, (m - d) * rtau)
    c2 = d * rtau + b_new
    E = jnp.exp(S * rtau[:, None] - c2[:, None])
    gwd = jnp.sum(E, axis=1)
    sES = jnp.sum(E * S, axis=1)
    g = gwd - jnp.exp(-b_new)
    P1 = sES - d * gwd
    ema = (1.0 - GAMMA) * s_old * jnp.exp(b_old - b_new) + GAMMA * g
    e0 = e0_ref[0, 0]
    sI = e0 * g + (1.0 - e0) * ema
    sIc = jnp.maximum(sI, EPS)
    rs = 1.0 / sIc
    loss_rows = P1 * rs
    wid_rows = loss_rows * rtau
    tw = jnp.log(sIc / (B - 1)) + b_new + RHO - wid_rows
    tw = jnp.clip(tw, -GRAD_CLIP, GRAD_CLIP)

    blk_loss = jnp.sum(loss_rows)
    blk_twsum = jnp.sum(tw)
    blk_twmax = jnp.max(tw)
    blk_twmin = jnp.min(tw)
    blk_tau = jnp.sum(tau)

    @pl.when((s == 0) & (i == 0))
    def _init():
        acc[0] = blk_loss
        acc[1] = blk_twsum
        acc[2] = 0.0
        acc[3] = blk_twmax
        acc[4] = blk_twmin
        acc[5] = blk_tau
        acc[6] = 0.0

    @pl.when((s != 0) | (i != 0))
    def _accum():
        acc[0] += blk_loss
        acc[1 + s] += blk_twsum
        acc[5 + s] += blk_tau

        @pl.when(s == 0)
        def _mm():
            acc[3] = jnp.maximum(acc[3], blk_twmax)
            acc[4] = jnp.minimum(acc[4], blk_twmin)

    @pl.when((s == 1) & (i == NB - 1))
    def _final():
        invB = jnp.float32(1.0 / B)
        loss_ref[0, 0] = acc[0] * invB
        taui_ref[0, 0] = acc[5] * invB
        taut_ref[0, 0] = acc[6] * invB
        twim_ref[0, 0] = acc[1] * invB
        twtm_ref[0, 0] = acc[2] * invB
        twmax_ref[0, 0] = acc[3]
        twmin_ref[0, 0] = acc[4]


_scal = jax.ShapeDtypeStruct((1, 1), jnp.float32)


def kernel(image_features, text_features, image_ids, text_ids, epoch, max_epoch,
           s_I, s_T, b_I, b_T, u_I, u_T, tau_I, tau_T, mask_neg):
    tau2, s2, b2 = _gather6(image_ids, text_ids, tau_I, s_I, b_I, tau_T, s_T, b_T)
    tau3 = tau2.reshape(2 * NB, 1, R)
    s3 = s2.reshape(2 * NB, 1, R)
    b3 = b2.reshape(2 * NB, 1, R)

    e0 = (jnp.asarray(epoch) == 0).astype(jnp.float32).reshape(1, 1)
    OT = jnp.concatenate([text_features, image_features], axis=1).T

    smem = pltpu.MemorySpace.SMEM
    outs = pl.pallas_call(
        _tc_body,
        grid=(2, NB),
        in_specs=[
            pl.BlockSpec((R, D), lambda s, i: (i, 0)),
            pl.BlockSpec((R, D), lambda s, i: (i, 0)),
            pl.BlockSpec((D, B), lambda s, i: (s, 0)),
            pl.BlockSpec((1, 1, R), lambda s, i: (s * NB + i, 0, 0)),
            pl.BlockSpec((1, 1, R), lambda s, i: (s * NB + i, 0, 0)),
            pl.BlockSpec((1, 1, R), lambda s, i: (s * NB + i, 0, 0)),
            pl.BlockSpec(memory_space=smem),
        ],
        out_specs=[pl.BlockSpec((1, 1), lambda s, i: (0, 0), memory_space=smem)] * 7,
        out_shape=[_scal] * 7,
        scratch_shapes=[pltpu.SMEM((8,), jnp.float32)],
    )(image_features, text_features, OT, tau3, s3, b3, e0)
    loss, taui, taut, twim, twtm, twmax, twmin = outs

    return (loss[0, 0], taui[0, 0], taut[0, 0], jnp.float32(ETA_INIT),
            twim[0, 0], twtm[0, 0], twmax[0, 0], twmin[0, 0])
